# Initial kernel scaffold; baseline (speedup 1.0000x reference)
import jax
import jax.numpy as jnp
from jax.experimental import pallas as pl


def kernel(x, edge_index, W_l, b_l, W_r):
    raise NotImplementedError("write your pallas kernel here")


# speedup vs baseline: 4.4998x; 4.4998x over previous
import functools

import jax
import jax.numpy as jnp
from jax import lax
from jax.experimental import pallas as pl
from jax.experimental.pallas import tpu as pltpu
from jax.experimental.pallas import tpu_sc as plsc

f32 = jnp.float32

_N = 10000
_E = 160000
_D = 256
_NP = 10240
_W = 144
_CH = 80
_NC, _NS = 2, 16
_EPT = _E // _NS
_NG = _EPT // _CH
_RPT = _NP // _NS


def _sc_aggregate(x01, src2, dst3):
  mesh = plsc.VectorSubcoreMesh(core_axis_name="c", subcore_axis_name="s")

  @functools.partial(
      pl.kernel,
      out_type=jax.ShapeDtypeStruct((_NC, _NP, _W), f32),
      mesh=mesh,
      scratch_types=[
          pltpu.VMEM((_NG, _CH), jnp.int32),
          pltpu.VMEM((_NG, _CH), jnp.int32),
          pltpu.VMEM((_CH, _W), f32),
          pltpu.VMEM_SHARED((_NP, _W), f32),
      ],
      compiler_params=pltpu.CompilerParams(use_tc_tiling_on_sc=False),
  )
  def body(x01_hbm, src_hbm, dst_hbm, out_hbm, srcv, dstv, rows, acc):
    cid = lax.axis_index("c")
    sid = lax.axis_index("s")

    pltpu.sync_copy(src_hbm.at[cid, sid], srcv)
    pltpu.sync_copy(dst_hbm.at[sid], dstv)

    zero = jnp.zeros((16,), f32)

    def zrows(k, c):
      rows[k // (_W // 16), pl.ds((k % (_W // 16)) * 16, 16)] = zero
      return c

    lax.fori_loop(0, _CH * (_W // 16), zrows, 0)

    def zacc(k, c):
      pltpu.sync_copy(rows, acc.at[pl.ds(sid * _RPT + k * _CH, _CH)])
      return c

    lax.fori_loop(0, _RPT // _CH, zacc, 0)
    plsc.subcore_barrier()

    def step(g, c):
      pltpu.sync_copy(x01_hbm.at[srcv.at[g]], rows)
      pltpu.sync_copy(rows, acc.at[dstv.at[g]], add=True)
      return c

    lax.fori_loop(0, _NG, step, 0)
    plsc.subcore_barrier()

    pltpu.sync_copy(acc.at[pl.ds(sid * _RPT, _RPT)],
                    out_hbm.at[cid, pl.ds(sid * _RPT, _RPT)])

  return body(x01, src2, dst3)


def _tc_dense(x, a0, a1, W_l, b_l, W_r):
  blk = 1000
  grid = (_N // blk,)

  def body(a0_ref, a1_ref, x_ref, wl_ref, wr_ref, b_ref, o_ref):
    a0b = a0_ref[...]
    a1b = a1_ref[...]
    xb = x_ref[...]
    cnt = a0b[:, 128:129]
    inv = 1.0 / jnp.maximum(cnt, 1.0)
    acc = jnp.dot(a0b[:, :128] * inv, wl_ref[0:128, :],
                  preferred_element_type=f32)
    acc = acc + jnp.dot(a1b[:, :128] * inv, wl_ref[128:256, :],
                        preferred_element_type=f32)
    acc = acc + jnp.dot(xb, wr_ref[...], preferred_element_type=f32)
    o_ref[...] = acc + xb + b_ref[...]

  return pl.pallas_call(
      body,
      grid=grid,
      in_specs=[
          pl.BlockSpec((blk, _W), lambda g: (g, 0)),
          pl.BlockSpec((blk, _W), lambda g: (g, 0)),
          pl.BlockSpec((blk, _D), lambda g: (g, 0)),
          pl.BlockSpec((_D, _D), lambda g: (0, 0)),
          pl.BlockSpec((_D, _D), lambda g: (0, 0)),
          pl.BlockSpec((1, _D), lambda g: (0, 0)),
      ],
      out_specs=pl.BlockSpec((blk, _D), lambda g: (g, 0)),
      out_shape=jax.ShapeDtypeStruct((_N, _D), f32),
  )(a0, a1, x, W_l, W_r, b_l.reshape(1, _D))


def kernel(x, edge_index, W_l, b_l, W_r):
  src = edge_index[0]
  dst = edge_index[1]
  x0 = jnp.concatenate(
      [x[:, :128], jnp.ones((_N, 1), f32), jnp.zeros((_N, 15), f32)], axis=1)
  x1 = jnp.concatenate([x[:, 128:], jnp.zeros((_N, 16), f32)], axis=1)
  x01 = jnp.concatenate([x0, x1], axis=0)
  src2 = jnp.stack([src, src + _N]).reshape(_NC, _NS, _NG, _CH)
  dst3 = dst.reshape(_NS, _NG, _CH)
  out01 = _sc_aggregate(x01, src2, dst3)
  return _tc_dense(x, out01[0], out01[1], W_l, b_l, W_r)


# speedup vs baseline: 5.3884x; 1.1975x over previous
import functools

import jax
import jax.numpy as jnp
from jax import lax
from jax.experimental import pallas as pl
from jax.experimental.pallas import tpu as pltpu
from jax.experimental.pallas import tpu_sc as plsc

f32 = jnp.float32

_N = 10000
_E = 160000
_D = 256
_NP = 10240
_W = 144
_CH = 80
_NC, _NS = 2, 16
_EPT = _E // _NS
_NG = _EPT // _CH
_RPT = _NP // _NS


def _sc_aggregate(x01, src2, dst3):
  mesh = plsc.VectorSubcoreMesh(core_axis_name="c", subcore_axis_name="s")

  @functools.partial(
      pl.kernel,
      out_type=jax.ShapeDtypeStruct((_NC, _NP, _W), f32),
      mesh=mesh,
      scratch_types=[
          pltpu.VMEM((3, _CH), jnp.int32),
          pltpu.VMEM((3, _CH), jnp.int32),
          pltpu.VMEM((2, _CH, _W), f32),
          pltpu.VMEM_SHARED((_NP, _W), f32),
          pltpu.SemaphoreType.DMA((3,)),
          pltpu.SemaphoreType.DMA((2,)),
      ],
      compiler_params=pltpu.CompilerParams(use_tc_tiling_on_sc=False),
  )
  def body(x01_hbm, src_hbm, dst_hbm, out_hbm, sidx, didx, rows2, acc,
           sem_i, sem_r):
    cid = lax.axis_index("c")
    sid = lax.axis_index("s")

    def idx_start(g):
      pltpu.async_copy(src_hbm.at[cid, sid, g], sidx.at[g % 3],
                       sem_i.at[g % 3])
      pltpu.async_copy(dst_hbm.at[sid, g], didx.at[g % 3], sem_i.at[g % 3])

    def idx_wait(g):
      pltpu.make_async_copy(src_hbm.at[cid, sid, g], sidx.at[g % 3],
                            sem_i.at[g % 3]).wait()
      pltpu.make_async_copy(dst_hbm.at[sid, g], didx.at[g % 3],
                            sem_i.at[g % 3]).wait()

    def gather_start(g):
      pltpu.async_copy(x01_hbm.at[sidx.at[g % 3]], rows2.at[g % 2],
                       sem_r.at[g % 2])

    def gather_wait(g):
      pltpu.make_async_copy(x01_hbm.at[sidx.at[g % 3]], rows2.at[g % 2],
                            sem_r.at[g % 2]).wait()

    idx_start(0)
    idx_start(1)

    zero = jnp.zeros((16,), f32)
    rows0 = rows2.at[0]

    def zrows(k, c):
      rows0[k // (_W // 16), pl.ds((k % (_W // 16)) * 16, 16)] = zero
      return c

    lax.fori_loop(0, _CH * (_W // 16), zrows, 0)

    def zacc(k, c):
      pltpu.sync_copy(rows0, acc.at[pl.ds(sid * _RPT + k * _CH, _CH)])
      return c

    lax.fori_loop(0, _RPT // _CH, zacc, 0)
    plsc.subcore_barrier()

    idx_wait(0)
    gather_start(0)

    def step(g, c):
      gather_wait(g)

      @pl.when(g + 2 < _NG)
      def _():
        idx_start(g + 2)

      @pl.when(g + 1 < _NG)
      def _():
        idx_wait(g + 1)
        gather_start(g + 1)

      pltpu.sync_copy(rows2.at[g % 2], acc.at[didx.at[g % 3]], add=True)
      return c

    lax.fori_loop(0, _NG, step, 0)
    plsc.subcore_barrier()

    pltpu.sync_copy(acc.at[pl.ds(sid * _RPT, _RPT)],
                    out_hbm.at[cid, pl.ds(sid * _RPT, _RPT)])

  return body(x01, src2, dst3)


def _tc_dense(x, a0, a1, W_l, b_l, W_r):
  blk = 1000
  grid = (_N // blk,)

  def body(a0_ref, a1_ref, x_ref, wl_ref, wr_ref, b_ref, o_ref):
    a0b = a0_ref[...]
    a1b = a1_ref[...]
    xb = x_ref[...]
    cnt = a0b[:, 128:129]
    inv = 1.0 / jnp.maximum(cnt, 1.0)
    acc = jnp.dot(a0b[:, :128] * inv, wl_ref[0:128, :],
                  preferred_element_type=f32)
    acc = acc + jnp.dot(a1b[:, :128] * inv, wl_ref[128:256, :],
                        preferred_element_type=f32)
    acc = acc + jnp.dot(xb, wr_ref[...], preferred_element_type=f32)
    o_ref[...] = acc + xb + b_ref[...]

  return pl.pallas_call(
      body,
      grid=grid,
      in_specs=[
          pl.BlockSpec((blk, _W), lambda g: (g, 0)),
          pl.BlockSpec((blk, _W), lambda g: (g, 0)),
          pl.BlockSpec((blk, _D), lambda g: (g, 0)),
          pl.BlockSpec((_D, _D), lambda g: (0, 0)),
          pl.BlockSpec((_D, _D), lambda g: (0, 0)),
          pl.BlockSpec((1, _D), lambda g: (0, 0)),
      ],
      out_specs=pl.BlockSpec((blk, _D), lambda g: (g, 0)),
      out_shape=jax.ShapeDtypeStruct((_N, _D), f32),
  )(a0, a1, x, W_l, W_r, b_l.reshape(1, _D))


def kernel(x, edge_index, W_l, b_l, W_r):
  src = edge_index[0]
  dst = edge_index[1]
  x0 = jnp.concatenate(
      [x[:, :128], jnp.ones((_N, 1), f32), jnp.zeros((_N, 15), f32)], axis=1)
  x1 = jnp.concatenate([x[:, 128:], jnp.zeros((_N, 16), f32)], axis=1)
  x01 = jnp.concatenate([x0, x1], axis=0)
  src2 = jnp.stack([src, src + _N]).reshape(_NC, _NS, _NG, _CH)
  dst3 = dst.reshape(_NS, _NG, _CH)
  out01 = _sc_aggregate(x01, src2, dst3)
  return _tc_dense(x, out01[0], out01[1], W_l, b_l, W_r)


# speedup vs baseline: 5.6399x; 1.0467x over previous
import functools

import jax
import jax.numpy as jnp
from jax import lax
from jax.experimental import pallas as pl
from jax.experimental.pallas import tpu as pltpu
from jax.experimental.pallas import tpu_sc as plsc

f32 = jnp.float32

_N = 10000
_E = 160000
_D = 256
_NP = 10240
_W = 144
_CH = 80
_NC, _NS = 2, 16
_EPT = _E // _NS
_NG = _EPT // _CH
_RPT = _NP // _NS


def _sc_aggregate(x01, src2, dst3):
  mesh = plsc.VectorSubcoreMesh(core_axis_name="c", subcore_axis_name="s")

  @functools.partial(
      pl.kernel,
      out_type=jax.ShapeDtypeStruct((_NC, _NP, _W), f32),
      mesh=mesh,
      scratch_types=[
          pltpu.VMEM((3, _CH), jnp.int32),
          pltpu.VMEM((3, _CH), jnp.int32),
          pltpu.VMEM((2, _CH, _W), f32),
          pltpu.VMEM_SHARED((_NP, _W), f32),
          pltpu.SemaphoreType.DMA((3,)),
          pltpu.SemaphoreType.DMA((2,)),
      ],
      compiler_params=pltpu.CompilerParams(use_tc_tiling_on_sc=False),
  )
  def body(x01_hbm, src_hbm, dst_hbm, out_hbm, sidx, didx, rows2, acc,
           sem_i, sem_r):
    cid = lax.axis_index("c")
    sid = lax.axis_index("s")

    def idx_start(g):
      pltpu.async_copy(src_hbm.at[cid, sid, g], sidx.at[g % 3],
                       sem_i.at[g % 3])
      pltpu.async_copy(dst_hbm.at[sid, g], didx.at[g % 3], sem_i.at[g % 3])

    def idx_wait(g):
      pltpu.make_async_copy(src_hbm.at[cid, sid, g], sidx.at[g % 3],
                            sem_i.at[g % 3]).wait()
      pltpu.make_async_copy(dst_hbm.at[sid, g], didx.at[g % 3],
                            sem_i.at[g % 3]).wait()

    def gather_start(g):
      pltpu.async_copy(x01_hbm.at[sidx.at[g % 3]], rows2.at[g % 2],
                       sem_r.at[g % 2])

    def gather_wait(g):
      pltpu.make_async_copy(x01_hbm.at[sidx.at[g % 3]], rows2.at[g % 2],
                            sem_r.at[g % 2]).wait()

    idx_start(0)
    idx_start(1)

    zero = jnp.zeros((16,), f32)
    rows0 = rows2.at[0]

    def zrows(k, c):
      rows0[k // (_W // 16), pl.ds((k % (_W // 16)) * 16, 16)] = zero
      return c

    lax.fori_loop(0, _CH * (_W // 16), zrows, 0)

    def zacc(k, c):
      pltpu.sync_copy(rows0, acc.at[pl.ds(sid * _RPT + k * _CH, _CH)])
      return c

    lax.fori_loop(0, _RPT // _CH, zacc, 0)
    plsc.subcore_barrier()

    idx_wait(0)
    gather_start(0)

    def step(g, c):
      gather_wait(g)

      @pl.when(g + 2 < _NG)
      def _():
        idx_start(g + 2)

      @pl.when(g + 1 < _NG)
      def _():
        idx_wait(g + 1)
        gather_start(g + 1)

      pltpu.sync_copy(rows2.at[g % 2], acc.at[didx.at[g % 3]], add=True)
      return c

    lax.fori_loop(0, _NG, step, 0)
    plsc.subcore_barrier()

    pltpu.sync_copy(acc.at[pl.ds(sid * _RPT, _RPT)],
                    out_hbm.at[cid, pl.ds(sid * _RPT, _RPT)])

  return body(x01, src2, dst3)


def _tc_self(x, W_r, b_l):
  blk = 1000
  grid = (_N // blk,)

  def body(x_ref, wr_ref, b_ref, o_ref):
    xb = x_ref[...]
    o_ref[...] = (jnp.dot(xb, wr_ref[...], preferred_element_type=f32)
                  + xb + b_ref[...])

  return pl.pallas_call(
      body,
      grid=grid,
      in_specs=[
          pl.BlockSpec((blk, _D), lambda g: (g, 0)),
          pl.BlockSpec((_D, _D), lambda g: (0, 0)),
          pl.BlockSpec((1, _D), lambda g: (0, 0)),
      ],
      out_specs=pl.BlockSpec((blk, _D), lambda g: (g, 0)),
      out_shape=jax.ShapeDtypeStruct((_N, _D), f32),
  )(x, W_r, b_l.reshape(1, _D))


def _tc_combine(out01, h, W_l):
  blk = 1000
  grid = (_N // blk,)

  def body(a0_ref, a1_ref, h_ref, wl_ref, o_ref):
    a0b = a0_ref[0]
    a1b = a1_ref[0]
    cnt = a0b[:, 128:129]
    inv = 1.0 / jnp.maximum(cnt, 1.0)
    acc = jnp.dot(a0b[:, :128] * inv, wl_ref[0:128, :],
                  preferred_element_type=f32)
    acc = acc + jnp.dot(a1b[:, :128] * inv, wl_ref[128:256, :],
                        preferred_element_type=f32)
    o_ref[...] = acc + h_ref[...]

  return pl.pallas_call(
      body,
      grid=grid,
      in_specs=[
          pl.BlockSpec((1, blk, _W), lambda g: (0, g, 0)),
          pl.BlockSpec((1, blk, _W), lambda g: (1, g, 0)),
          pl.BlockSpec((blk, _D), lambda g: (g, 0)),
          pl.BlockSpec((_D, _D), lambda g: (0, 0)),
      ],
      out_specs=pl.BlockSpec((blk, _D), lambda g: (g, 0)),
      out_shape=jax.ShapeDtypeStruct((_N, _D), f32),
  )(out01, out01, h, W_l)


def kernel(x, edge_index, W_l, b_l, W_r):
  src = edge_index[0]
  dst = edge_index[1]
  x0 = jnp.concatenate(
      [x[:, :128], jnp.ones((_N, 1), f32), jnp.zeros((_N, 15), f32)], axis=1)
  x1 = jnp.concatenate([x[:, 128:], jnp.zeros((_N, 16), f32)], axis=1)
  x01 = jnp.concatenate([x0, x1], axis=0)
  src2 = jnp.stack([src, src + _N]).reshape(_NC, _NS, _NG, _CH)
  dst3 = dst.reshape(_NS, _NG, _CH)
  out01 = _sc_aggregate(x01, src2, dst3)
  h = _tc_self(x, W_r, b_l)
  return _tc_combine(out01, h, W_l)


# speedup vs baseline: 7.1207x; 1.2625x over previous
import functools

import jax
import jax.numpy as jnp
from jax import lax
from jax.experimental import pallas as pl
from jax.experimental.pallas import tpu as pltpu
from jax.experimental.pallas import tpu_sc as plsc

f32 = jnp.float32

_N = 10000
_E = 160000
_D = 256
_NP = 10240
_W = 128
_CH = 80
_NC, _NS = 2, 16
_EPT = _E // _NS
_NG = _EPT // _CH
_RPT = _NP // _NS


def _sc_aggregate(x01, src2, dst3):
  mesh = plsc.VectorSubcoreMesh(core_axis_name="c", subcore_axis_name="s")

  @functools.partial(
      pl.kernel,
      out_type=[jax.ShapeDtypeStruct((_NC, _NP, _W), f32),
                jax.ShapeDtypeStruct((_NC, _NP, 16), f32)],
      mesh=mesh,
      scratch_types=[
          pltpu.VMEM((3, _CH), jnp.int32),
          pltpu.VMEM((3, _CH), jnp.int32),
          pltpu.VMEM((2, _CH, _W), f32),
          pltpu.VMEM((_CH, 16), f32),
          pltpu.VMEM_SHARED((_NP, _W), f32),
          pltpu.VMEM_SHARED((_NP, 16), f32),
          pltpu.SemaphoreType.DMA((3,)),
          pltpu.SemaphoreType.DMA((2,)),
      ],
      compiler_params=pltpu.CompilerParams(use_tc_tiling_on_sc=False),
  )
  def body(x01_hbm, src_hbm, dst_hbm, out_hbm, outc_hbm, sidx, didx, rows2,
           ones, acc, cnt, sem_i, sem_r):
    cid = lax.axis_index("c")
    sid = lax.axis_index("s")

    def idx_start(g):
      pltpu.async_copy(src_hbm.at[cid, sid, g], sidx.at[g % 3],
                       sem_i.at[g % 3])
      pltpu.async_copy(dst_hbm.at[sid, g], didx.at[g % 3], sem_i.at[g % 3])

    def idx_wait(g):
      pltpu.make_async_copy(src_hbm.at[cid, sid, g], sidx.at[g % 3],
                            sem_i.at[g % 3]).wait()
      pltpu.make_async_copy(dst_hbm.at[sid, g], didx.at[g % 3],
                            sem_i.at[g % 3]).wait()

    def gather_start(g):
      pltpu.async_copy(x01_hbm.at[sidx.at[g % 3]], rows2.at[g % 2],
                       sem_r.at[g % 2])

    def gather_wait(g):
      pltpu.make_async_copy(x01_hbm.at[sidx.at[g % 3]], rows2.at[g % 2],
                            sem_r.at[g % 2]).wait()

    idx_start(0)
    idx_start(1)

    zero = jnp.zeros((16,), f32)
    rows0 = rows2.at[0]

    def zrows(k, c):
      rows0[k // (_W // 16), pl.ds((k % (_W // 16)) * 16, 16)] = zero
      return c

    lax.fori_loop(0, _CH * (_W // 16), zrows, 0)

    def zones(k, c):
      ones[k, pl.ds(0, 16)] = zero
      return c

    lax.fori_loop(0, _CH, zones, 0)

    def zacc(k, c):
      pltpu.sync_copy(rows0, acc.at[pl.ds(sid * _RPT + k * _CH, _CH)])
      return c

    lax.fori_loop(0, _RPT // _CH, zacc, 0)

    def zcnt(k, c):
      pltpu.sync_copy(ones, cnt.at[pl.ds(sid * _RPT + k * _CH, _CH)])
      return c

    lax.fori_loop(0, _RPT // _CH, zcnt, 0)

    one = jnp.ones((16,), f32)

    def fones(k, c):
      ones[k, pl.ds(0, 16)] = one
      return c

    lax.fori_loop(0, _CH, fones, 0)
    plsc.subcore_barrier()

    idx_wait(0)
    gather_start(0)

    def step(g, c):
      gather_wait(g)

      @pl.when(g + 2 < _NG)
      def _():
        idx_start(g + 2)

      @pl.when(g + 1 < _NG)
      def _():
        idx_wait(g + 1)
        gather_start(g + 1)

      pltpu.sync_copy(rows2.at[g % 2], acc.at[didx.at[g % 3]], add=True)

      @pl.when(g % 2 == cid)
      def _():
        pltpu.sync_copy(ones, cnt.at[didx.at[g % 3]], add=True)

      return c

    lax.fori_loop(0, _NG, step, 0)
    plsc.subcore_barrier()

    pltpu.sync_copy(acc.at[pl.ds(sid * _RPT, _RPT)],
                    out_hbm.at[cid, pl.ds(sid * _RPT, _RPT)])
    pltpu.sync_copy(cnt.at[pl.ds(sid * _RPT, _RPT)],
                    outc_hbm.at[cid, pl.ds(sid * _RPT, _RPT)])

  return body(x01, src2, dst3)


def _tc_dense(x, out01, outc, W_l, b_l, W_r):
  blk = 1000
  grid = (_N // blk,)

  def body(a0_ref, a1_ref, c0_ref, c1_ref, x_ref, wl_ref, wr_ref, b_ref,
           o_ref):
    a0b = a0_ref[0]
    a1b = a1_ref[0]
    cnt = c0_ref[0][:, 0:1] + c1_ref[0][:, 0:1]
    inv = 1.0 / jnp.maximum(cnt, 1.0)
    xb = x_ref[...]
    acc = jnp.dot(a0b * inv, wl_ref[0:128, :], preferred_element_type=f32)
    acc = acc + jnp.dot(a1b * inv, wl_ref[128:256, :],
                        preferred_element_type=f32)
    acc = acc + jnp.dot(xb, wr_ref[...], preferred_element_type=f32)
    o_ref[...] = acc + xb + b_ref[...]

  return pl.pallas_call(
      body,
      grid=grid,
      in_specs=[
          pl.BlockSpec((1, blk, _W), lambda g: (0, g, 0)),
          pl.BlockSpec((1, blk, _W), lambda g: (1, g, 0)),
          pl.BlockSpec((1, blk, 16), lambda g: (0, g, 0)),
          pl.BlockSpec((1, blk, 16), lambda g: (1, g, 0)),
          pl.BlockSpec((blk, _D), lambda g: (g, 0)),
          pl.BlockSpec((_D, _D), lambda g: (0, 0)),
          pl.BlockSpec((_D, _D), lambda g: (0, 0)),
          pl.BlockSpec((1, _D), lambda g: (0, 0)),
      ],
      out_specs=pl.BlockSpec((blk, _D), lambda g: (g, 0)),
      out_shape=jax.ShapeDtypeStruct((_N, _D), f32),
  )(out01, out01, outc, outc, x, W_l, W_r, b_l.reshape(1, _D))


def kernel(x, edge_index, W_l, b_l, W_r):
  src = edge_index[0]
  dst = edge_index[1]
  x01 = x.reshape(_N, 2, _W).transpose(1, 0, 2).reshape(2 * _N, _W)
  src2 = jnp.stack([src, src + _N]).reshape(_NC, _NS, _NG, _CH)
  dst3 = dst.reshape(_NS, _NG, _CH)
  out01, outc = _sc_aggregate(x01, src2, dst3)
  return _tc_dense(x, out01, outc, W_l, b_l, W_r)
